# baseline (speedup 1.0000x reference)
import functools

import jax
import jax.numpy as jnp
from jax import lax
from jax.experimental import pallas as pl
from jax.experimental.pallas import tpu as pltpu
from jax.experimental.pallas import tpu_sc as plsc

_N = 10000
_E = 320000
_D = 128
_G = 64
_OUT = 16

_HD = 64
_NCH = 500
_K = 40
_WR = 624
_ZROWS = 104


_RING = 5


def _sc_scatter_body(h0_hbm, h1_hbm, src_hbm, dst_hbm, out_hbm,
                     idx_s, idx_d, rows, zbuf, acc, *all_sems):
    gsems = all_sems[:2 * _RING]
    ssems = all_sems[2 * _RING:]
    c = lax.axis_index("c")
    s = lax.axis_index("s")

    pltpu.sync_copy(src_hbm.at[s], idx_s)
    pltpu.sync_copy(dst_hbm.at[s], idx_d)

    def zbody(i, carry):
        for jj in range(_HD // 16):
            zbuf[i, pl.ds(jj * 16, 16)] = jnp.zeros((16,), jnp.float32)
        return carry
    lax.fori_loop(0, _ZROWS, zbody, 0)
    base = s * _WR
    for r in range(_WR // _ZROWS):
        pltpu.sync_copy(zbuf, acc.at[pl.ds(base + r * _ZROWS, _ZROWS)])

    @pl.when(s == 15)
    def _ztail():
        pltpu.sync_copy(zbuf.at[pl.ds(0, _N - 16 * _WR)],
                        acc.at[pl.ds(16 * _WR, _N - 16 * _WR)])

    plsc.subcore_barrier()

    nb_tot = 2 * _RING

    def _run(h_hbm):
        for b in range(_RING):
            pltpu.async_copy(h_hbm.at[idx_s.at[b]], rows.at[b], gsems[b])

        def body(gq, carry):
            for u in range(nb_tot):
                j = gq * nb_tot + u
                pltpu.make_async_copy(h_hbm.at[idx_s.at[j]], rows.at[u],
                                      gsems[u]).wait()
                pltpu.async_copy(rows.at[u], acc.at[idx_d.at[j]], ssems[u],
                                 add=True)
                nxt = (u + _RING) % nb_tot
                nj = j + _RING

                @pl.when(nj < _NCH)
                def _refill():
                    @pl.when(j >= _RING)
                    def _drain_old():
                        pltpu.make_async_copy(
                            rows.at[nxt], acc.at[idx_d.at[nj - nb_tot]],
                            ssems[nxt]).wait()
                    pltpu.async_copy(h_hbm.at[idx_s.at[nj]], rows.at[nxt],
                                     gsems[nxt])
            return carry
        lax.fori_loop(0, _NCH // nb_tot, body, 0)

        for u in range(nb_tot):
            pltpu.make_async_copy(rows.at[u],
                                  acc.at[idx_d.at[_NCH - nb_tot + u]],
                                  ssems[u]).wait()

    @pl.when(c == 0)
    def _c0():
        _run(h0_hbm)

    @pl.when(c == 1)
    def _c1():
        _run(h1_hbm)

    plsc.subcore_barrier()

    pltpu.sync_copy(acc.at[pl.ds(base, _WR)], out_hbm.at[c, pl.ds(base, _WR)])

    @pl.when(s == 15)
    def _wtail():
        pltpu.sync_copy(acc.at[pl.ds(16 * _WR, _N - 16 * _WR)],
                        out_hbm.at[c, pl.ds(16 * _WR, _N - 16 * _WR)])


def _make_sc_scatter():
    mesh = plsc.VectorSubcoreMesh(core_axis_name="c", subcore_axis_name="s")
    return pl.kernel(
        _sc_scatter_body,
        mesh=mesh,
        compiler_params=pltpu.CompilerParams(use_tc_tiling_on_sc=False),
        out_type=jax.ShapeDtypeStruct((2, _N, _HD), jnp.float32),
        scratch_types=[
            pltpu.VMEM((_NCH, _K), jnp.int32),
            pltpu.VMEM((_NCH, _K), jnp.int32),
            pltpu.VMEM((2 * _RING, _K, _HD), jnp.float32),
            pltpu.VMEM((_ZROWS, _HD), jnp.float32),
            pltpu.VMEM_SHARED((_N, _HD), jnp.float32),
        ] + [pltpu.SemaphoreType.DMA] * (4 * _RING),
    )


_BLK = 400
_NBLK = _N // _BLK


def _mlp1_body(hlo_ref, hhi_ref, p_ref, w1_ref, b1_ref, h1_ref, s1_ref,
               s2_ref, sm2_ref):
    i = pl.program_id(0)
    z = jnp.concatenate([hlo_ref[...] + p_ref[0], hhi_ref[...] + p_ref[1]],
                        axis=-1)
    h1 = jnp.dot(z, w1_ref[...], preferred_element_type=jnp.float32) + b1_ref[...]
    h1_ref[...] = h1

    @pl.when(i == 0)
    def _init():
        s1_ref[...] = jnp.zeros_like(s1_ref)
        s2_ref[...] = jnp.zeros_like(s2_ref)
        sm2_ref[...] = jnp.zeros_like(sm2_ref)

    mb = jnp.mean(h1, axis=0, keepdims=True)
    d = h1 - mb
    s1_ref[...] += mb
    s2_ref[...] += mb * mb
    sm2_ref[...] += jnp.sum(d * d, axis=0, keepdims=True)


def _mlp1(hlo, hhi, part, w1, b1r):
    return pl.pallas_call(
        _mlp1_body,
        grid=(_NBLK,),
        in_specs=[
            pl.BlockSpec((_BLK, _HD), lambda i: (i, 0)),
            pl.BlockSpec((_BLK, _HD), lambda i: (i, 0)),
            pl.BlockSpec((2, _BLK, _HD), lambda i: (0, i, 0)),
            pl.BlockSpec((_D, _D), lambda i: (0, 0)),
            pl.BlockSpec((1, _D), lambda i: (0, 0)),
        ],
        out_specs=[
            pl.BlockSpec((_BLK, _D), lambda i: (i, 0)),
            pl.BlockSpec((1, _D), lambda i: (0, 0)),
            pl.BlockSpec((1, _D), lambda i: (0, 0)),
            pl.BlockSpec((1, _D), lambda i: (0, 0)),
        ],
        out_shape=[
            jax.ShapeDtypeStruct((_N, _D), jnp.float32),
            jax.ShapeDtypeStruct((1, _D), jnp.float32),
            jax.ShapeDtypeStruct((1, _D), jnp.float32),
            jax.ShapeDtypeStruct((1, _D), jnp.float32),
        ],
    )(hlo, hhi, part, w1, b1r)


def _mlp2_body(h1_ref, s1_ref, s2_ref, sm2_ref, g_ref, be_ref, w2_ref, b2_ref,
               batch_ref, hlo_ref, hhi_ref, pooled_ref):
    i = pl.program_id(0)
    m = s1_ref[...] * (1.0 / _NBLK)
    spread = s2_ref[...] - _NBLK * m * m
    v = (sm2_ref[...] + _BLK * spread) * (1.0 / _N)
    inv = lax.rsqrt(v + 1e-5) * g_ref[...]
    r = jnp.maximum((h1_ref[...] - m) * inv + be_ref[...], 0.0)
    hout = jnp.dot(r, w2_ref[...], preferred_element_type=jnp.float32) + b2_ref[...]
    hlo_ref[...] = hout[:, :_HD]
    hhi_ref[...] = hout[:, _HD:]

    gid = lax.broadcasted_iota(jnp.int32, (_G, _BLK), 0)
    onehot = (gid == batch_ref[0]).astype(jnp.float32)

    @pl.when(i == 0)
    def _init():
        pooled_ref[...] = jnp.zeros_like(pooled_ref)

    pooled_ref[...] += jnp.dot(onehot, hout, preferred_element_type=jnp.float32, precision=lax.Precision.HIGHEST)


def _mlp2(h1, s1, s2, sm2, gr, ber, w2, b2r, batch3d):
    return pl.pallas_call(
        _mlp2_body,
        grid=(_NBLK,),
        in_specs=[
            pl.BlockSpec((_BLK, _D), lambda i: (i, 0)),
            pl.BlockSpec((1, _D), lambda i: (0, 0)),
            pl.BlockSpec((1, _D), lambda i: (0, 0)),
            pl.BlockSpec((1, _D), lambda i: (0, 0)),
            pl.BlockSpec((1, _D), lambda i: (0, 0)),
            pl.BlockSpec((1, _D), lambda i: (0, 0)),
            pl.BlockSpec((_D, _D), lambda i: (0, 0)),
            pl.BlockSpec((1, _D), lambda i: (0, 0)),
            pl.BlockSpec((1, 1, _BLK), lambda i: (i, 0, 0)),
        ],
        out_specs=[
            pl.BlockSpec((_BLK, _HD), lambda i: (i, 0)),
            pl.BlockSpec((_BLK, _HD), lambda i: (i, 0)),
            pl.BlockSpec((_G, _D), lambda i: (0, 0)),
        ],
        out_shape=[
            jax.ShapeDtypeStruct((_N, _HD), jnp.float32),
            jax.ShapeDtypeStruct((_N, _HD), jnp.float32),
            jax.ShapeDtypeStruct((_G, _D), jnp.float32),
        ],
    )(h1, s1, s2, sm2, gr, ber, w2, b2r, batch3d)


def _head_body(p0_ref, p1_ref, p2_ref, wa_ref, wb_ref, wc_ref, bp1_ref,
               gp_ref, bep_ref, wp2_ref, bp2_ref, out_ref):
    t = (jnp.dot(p0_ref[...], wa_ref[...], preferred_element_type=jnp.float32)
         + jnp.dot(p1_ref[...], wb_ref[...], preferred_element_type=jnp.float32)
         + jnp.dot(p2_ref[...], wc_ref[...], preferred_element_type=jnp.float32)
         + bp1_ref[...])
    m = jnp.mean(t, axis=0, keepdims=True)
    d = t - m
    v = jnp.mean(d * d, axis=0, keepdims=True)
    r = jnp.maximum(d * lax.rsqrt(v + 1e-5) * gp_ref[...] + bep_ref[...], 0.0)
    out_ref[...] = jnp.dot(r, wp2_ref[...], preferred_element_type=jnp.float32) + bp2_ref[...]


def _head(p0, p1, p2, wa, wb, wc, bp1r, gpr, bepr, wp2p, bp2p):
    return pl.pallas_call(
        _head_body,
        out_shape=jax.ShapeDtypeStruct((_G, _D), jnp.float32),
    )(p0, p1, p2, wa, wb, wc, bp1r, gpr, bepr, wp2p, bp2p)


def kernel(x, edge_index, batch, W1_0, b1_0, g_0, be_0, W2_0, b2_0,
           W1_1, b1_1, g_1, be_1, W2_1, b2_1, W1_2, b1_2, g_2, be_2,
           W2_2, b2_2, Wp1, bp1, gp, bep, Wp2, bp2):
    src_r = edge_index[0].reshape(16, _NCH, _K)
    dst_r = edge_index[1].reshape(16, _NCH, _K)
    assert _NCH % (2 * _RING) == 0
    batch3d = batch.reshape(_NBLK, 1, _BLK)

    sc_scatter = _make_sc_scatter()

    params = [
        (W1_0, b1_0, g_0, be_0, W2_0, b2_0),
        (W1_1, b1_1, g_1, be_1, W2_1, b2_1),
        (W1_2, b1_2, g_2, be_2, W2_2, b2_2),
    ]
    hlo, hhi = x[:, :_HD], x[:, _HD:]
    pooled = []
    for (w1, b1, g, be, w2, b2) in params:
        part = sc_scatter(hlo, hhi, src_r, dst_r)
        h1, s1, s2, sm2 = _mlp1(hlo, hhi, part, w1, b1.reshape(1, _D))
        hlo, hhi, pool = _mlp2(h1, s1, s2, sm2, g.reshape(1, _D),
                               be.reshape(1, _D), w2, b2.reshape(1, _D),
                               batch3d)
        pooled.append(pool)

    wa = Wp1[0:_D]
    wb = Wp1[_D:2 * _D]
    wc = Wp1[2 * _D:3 * _D]
    wp2p = jnp.pad(Wp2, ((0, 0), (0, _D - _OUT)))
    bp2p = jnp.pad(bp2, (0, _D - _OUT)).reshape(1, _D)
    out = _head(pooled[0], pooled[1], pooled[2], wa, wb, wc,
                bp1.reshape(1, _D), gp.reshape(1, _D), bep.reshape(1, _D),
                wp2p, bp2p)
    return out[:, :_OUT]


# speedup vs baseline: 1.0516x; 1.0516x over previous
import functools

import jax
import jax.numpy as jnp
from jax import lax
from jax.experimental import pallas as pl
from jax.experimental.pallas import tpu as pltpu
from jax.experimental.pallas import tpu_sc as plsc

_N = 10000
_E = 320000
_D = 128
_G = 64
_OUT = 16

_HD = 64
_NCH = 250
_K = 80
_WR = 624
_ZROWS = 104


_NBUF = 5
_RG = 3


def _sc_scatter_body(h0_hbm, h1_hbm, src_hbm, dst_hbm, out_hbm,
                     idx_s, idx_d, rows, zbuf, acc, *all_sems):
    gsems = all_sems[:_NBUF]
    ssems = all_sems[_NBUF:]
    c = lax.axis_index("c")
    s = lax.axis_index("s")

    pltpu.sync_copy(src_hbm.at[s], idx_s)
    pltpu.sync_copy(dst_hbm.at[s], idx_d)

    def zbody(i, carry):
        for jj in range(_HD // 16):
            zbuf[i, pl.ds(jj * 16, 16)] = jnp.zeros((16,), jnp.float32)
        return carry
    lax.fori_loop(0, _ZROWS, zbody, 0)
    base = s * _WR
    for r in range(_WR // _ZROWS):
        pltpu.sync_copy(zbuf, acc.at[pl.ds(base + r * _ZROWS, _ZROWS)])

    @pl.when(s == 15)
    def _ztail():
        pltpu.sync_copy(zbuf.at[pl.ds(0, _N - 16 * _WR)],
                        acc.at[pl.ds(16 * _WR, _N - 16 * _WR)])

    plsc.subcore_barrier()

    def _run(h_hbm):
        for b in range(_RG):
            pltpu.async_copy(h_hbm.at[idx_s.at[b]], rows.at[b], gsems[b])

        def body(gq, carry):
            for u in range(_NBUF):
                j = gq * _NBUF + u
                pltpu.make_async_copy(h_hbm.at[idx_s.at[j]], rows.at[u],
                                      gsems[u]).wait()
                pltpu.async_copy(rows.at[u], acc.at[idx_d.at[j]], ssems[u],
                                 add=True)
                nxt = (u + _RG) % _NBUF
                nj = j + _RG

                @pl.when(nj < _NCH)
                def _refill():
                    @pl.when(j >= _NBUF - _RG)
                    def _drain_old():
                        pltpu.make_async_copy(
                            rows.at[nxt], acc.at[idx_d.at[nj - _NBUF]],
                            ssems[nxt]).wait()
                    pltpu.async_copy(h_hbm.at[idx_s.at[nj]], rows.at[nxt],
                                     gsems[nxt])
            return carry
        lax.fori_loop(0, _NCH // _NBUF, body, 0)

        for u in range(_NBUF):
            pltpu.make_async_copy(rows.at[u],
                                  acc.at[idx_d.at[_NCH - _NBUF + u]],
                                  ssems[u]).wait()

    @pl.when(c == 0)
    def _c0():
        _run(h0_hbm)

    @pl.when(c == 1)
    def _c1():
        _run(h1_hbm)

    plsc.subcore_barrier()

    pltpu.sync_copy(acc.at[pl.ds(base, _WR)], out_hbm.at[c, pl.ds(base, _WR)])

    @pl.when(s == 15)
    def _wtail():
        pltpu.sync_copy(acc.at[pl.ds(16 * _WR, _N - 16 * _WR)],
                        out_hbm.at[c, pl.ds(16 * _WR, _N - 16 * _WR)])


def _make_sc_scatter():
    mesh = plsc.VectorSubcoreMesh(core_axis_name="c", subcore_axis_name="s")
    return pl.kernel(
        _sc_scatter_body,
        mesh=mesh,
        compiler_params=pltpu.CompilerParams(use_tc_tiling_on_sc=False),
        out_type=jax.ShapeDtypeStruct((2, _N, _HD), jnp.float32),
        scratch_types=[
            pltpu.VMEM((_NCH, _K), jnp.int32),
            pltpu.VMEM((_NCH, _K), jnp.int32),
            pltpu.VMEM((_NBUF, _K, _HD), jnp.float32),
            pltpu.VMEM((_ZROWS, _HD), jnp.float32),
            pltpu.VMEM_SHARED((_N, _HD), jnp.float32),
        ] + [pltpu.SemaphoreType.DMA] * (2 * _NBUF),
    )


_BLK = 400
_NBLK = _N // _BLK


def _mlp1_body(hlo_ref, hhi_ref, p_ref, w1_ref, b1_ref, h1_ref, s1_ref,
               s2_ref, sm2_ref):
    i = pl.program_id(0)
    z = jnp.concatenate([hlo_ref[...] + p_ref[0], hhi_ref[...] + p_ref[1]],
                        axis=-1)
    h1 = jnp.dot(z, w1_ref[...], preferred_element_type=jnp.float32) + b1_ref[...]
    h1_ref[...] = h1

    @pl.when(i == 0)
    def _init():
        s1_ref[...] = jnp.zeros_like(s1_ref)
        s2_ref[...] = jnp.zeros_like(s2_ref)
        sm2_ref[...] = jnp.zeros_like(sm2_ref)

    mb = jnp.mean(h1, axis=0, keepdims=True)
    d = h1 - mb
    s1_ref[...] += mb
    s2_ref[...] += mb * mb
    sm2_ref[...] += jnp.sum(d * d, axis=0, keepdims=True)


def _mlp1(hlo, hhi, part, w1, b1r):
    return pl.pallas_call(
        _mlp1_body,
        grid=(_NBLK,),
        in_specs=[
            pl.BlockSpec((_BLK, _HD), lambda i: (i, 0)),
            pl.BlockSpec((_BLK, _HD), lambda i: (i, 0)),
            pl.BlockSpec((2, _BLK, _HD), lambda i: (0, i, 0)),
            pl.BlockSpec((_D, _D), lambda i: (0, 0)),
            pl.BlockSpec((1, _D), lambda i: (0, 0)),
        ],
        out_specs=[
            pl.BlockSpec((_BLK, _D), lambda i: (i, 0)),
            pl.BlockSpec((1, _D), lambda i: (0, 0)),
            pl.BlockSpec((1, _D), lambda i: (0, 0)),
            pl.BlockSpec((1, _D), lambda i: (0, 0)),
        ],
        out_shape=[
            jax.ShapeDtypeStruct((_N, _D), jnp.float32),
            jax.ShapeDtypeStruct((1, _D), jnp.float32),
            jax.ShapeDtypeStruct((1, _D), jnp.float32),
            jax.ShapeDtypeStruct((1, _D), jnp.float32),
        ],
    )(hlo, hhi, part, w1, b1r)


def _mlp2_body(h1_ref, s1_ref, s2_ref, sm2_ref, g_ref, be_ref, w2_ref, b2_ref,
               batch_ref, hlo_ref, hhi_ref, pooled_ref):
    i = pl.program_id(0)
    m = s1_ref[...] * (1.0 / _NBLK)
    spread = s2_ref[...] - _NBLK * m * m
    v = (sm2_ref[...] + _BLK * spread) * (1.0 / _N)
    inv = lax.rsqrt(v + 1e-5) * g_ref[...]
    r = jnp.maximum((h1_ref[...] - m) * inv + be_ref[...], 0.0)
    hout = jnp.dot(r, w2_ref[...], preferred_element_type=jnp.float32) + b2_ref[...]
    hlo_ref[...] = hout[:, :_HD]
    hhi_ref[...] = hout[:, _HD:]

    gid = lax.broadcasted_iota(jnp.int32, (_G, _BLK), 0)
    onehot = (gid == batch_ref[0]).astype(jnp.float32)

    @pl.when(i == 0)
    def _init():
        pooled_ref[...] = jnp.zeros_like(pooled_ref)

    pooled_ref[...] += jnp.dot(onehot, hout, preferred_element_type=jnp.float32, precision=lax.Precision.HIGHEST)


def _mlp2(h1, s1, s2, sm2, gr, ber, w2, b2r, batch3d):
    return pl.pallas_call(
        _mlp2_body,
        grid=(_NBLK,),
        in_specs=[
            pl.BlockSpec((_BLK, _D), lambda i: (i, 0)),
            pl.BlockSpec((1, _D), lambda i: (0, 0)),
            pl.BlockSpec((1, _D), lambda i: (0, 0)),
            pl.BlockSpec((1, _D), lambda i: (0, 0)),
            pl.BlockSpec((1, _D), lambda i: (0, 0)),
            pl.BlockSpec((1, _D), lambda i: (0, 0)),
            pl.BlockSpec((_D, _D), lambda i: (0, 0)),
            pl.BlockSpec((1, _D), lambda i: (0, 0)),
            pl.BlockSpec((1, 1, _BLK), lambda i: (i, 0, 0)),
        ],
        out_specs=[
            pl.BlockSpec((_BLK, _HD), lambda i: (i, 0)),
            pl.BlockSpec((_BLK, _HD), lambda i: (i, 0)),
            pl.BlockSpec((_G, _D), lambda i: (0, 0)),
        ],
        out_shape=[
            jax.ShapeDtypeStruct((_N, _HD), jnp.float32),
            jax.ShapeDtypeStruct((_N, _HD), jnp.float32),
            jax.ShapeDtypeStruct((_G, _D), jnp.float32),
        ],
    )(h1, s1, s2, sm2, gr, ber, w2, b2r, batch3d)


def _head_body(p0_ref, p1_ref, p2_ref, wa_ref, wb_ref, wc_ref, bp1_ref,
               gp_ref, bep_ref, wp2_ref, bp2_ref, out_ref):
    t = (jnp.dot(p0_ref[...], wa_ref[...], preferred_element_type=jnp.float32)
         + jnp.dot(p1_ref[...], wb_ref[...], preferred_element_type=jnp.float32)
         + jnp.dot(p2_ref[...], wc_ref[...], preferred_element_type=jnp.float32)
         + bp1_ref[...])
    m = jnp.mean(t, axis=0, keepdims=True)
    d = t - m
    v = jnp.mean(d * d, axis=0, keepdims=True)
    r = jnp.maximum(d * lax.rsqrt(v + 1e-5) * gp_ref[...] + bep_ref[...], 0.0)
    out_ref[...] = jnp.dot(r, wp2_ref[...], preferred_element_type=jnp.float32) + bp2_ref[...]


def _head(p0, p1, p2, wa, wb, wc, bp1r, gpr, bepr, wp2p, bp2p):
    return pl.pallas_call(
        _head_body,
        out_shape=jax.ShapeDtypeStruct((_G, _D), jnp.float32),
    )(p0, p1, p2, wa, wb, wc, bp1r, gpr, bepr, wp2p, bp2p)


def kernel(x, edge_index, batch, W1_0, b1_0, g_0, be_0, W2_0, b2_0,
           W1_1, b1_1, g_1, be_1, W2_1, b2_1, W1_2, b1_2, g_2, be_2,
           W2_2, b2_2, Wp1, bp1, gp, bep, Wp2, bp2):
    src_r = edge_index[0].reshape(16, _NCH, _K)
    dst_r = edge_index[1].reshape(16, _NCH, _K)
    assert _NCH % _NBUF == 0
    batch3d = batch.reshape(_NBLK, 1, _BLK)

    sc_scatter = _make_sc_scatter()

    params = [
        (W1_0, b1_0, g_0, be_0, W2_0, b2_0),
        (W1_1, b1_1, g_1, be_1, W2_1, b2_1),
        (W1_2, b1_2, g_2, be_2, W2_2, b2_2),
    ]
    hlo, hhi = x[:, :_HD], x[:, _HD:]
    pooled = []
    for (w1, b1, g, be, w2, b2) in params:
        part = sc_scatter(hlo, hhi, src_r, dst_r)
        h1, s1, s2, sm2 = _mlp1(hlo, hhi, part, w1, b1.reshape(1, _D))
        hlo, hhi, pool = _mlp2(h1, s1, s2, sm2, g.reshape(1, _D),
                               be.reshape(1, _D), w2, b2.reshape(1, _D),
                               batch3d)
        pooled.append(pool)

    wa = Wp1[0:_D]
    wb = Wp1[_D:2 * _D]
    wc = Wp1[2 * _D:3 * _D]
    wp2p = jnp.pad(Wp2, ((0, 0), (0, _D - _OUT)))
    bp2p = jnp.pad(bp2, (0, _D - _OUT)).reshape(1, _D)
    out = _head(pooled[0], pooled[1], pooled[2], wa, wb, wc,
                bp1.reshape(1, _D), gp.reshape(1, _D), bep.reshape(1, _D),
                wp2p, bp2p)
    return out[:, :_OUT]


# speedup vs baseline: 1.2852x; 1.2221x over previous
import functools

import jax
import jax.numpy as jnp
from jax import lax
from jax.experimental import pallas as pl
from jax.experimental.pallas import tpu as pltpu
from jax.experimental.pallas import tpu_sc as plsc

_N = 10000
_E = 320000
_D = 128
_G = 64
_OUT = 16

_HD = 64
_NCH = 250
_K = 40
_WR = 624
_ZROWS = 24


_RING = 5


def _sc_scatter_body(h_hbm, src_hbm, dst_hbm, out_hbm,
                     idx_s, idx_d, rows, zbuf, acc, *gsems):
    c = lax.axis_index("c")
    s = lax.axis_index("s")

    pltpu.sync_copy(src_hbm.at[c, s], idx_s)
    pltpu.sync_copy(dst_hbm.at[c, s], idx_d)

    def zbody(i, carry):
        for jj in range(_D // 16):
            zbuf[i, pl.ds(jj * 16, 16)] = jnp.zeros((16,), jnp.float32)
        return carry
    lax.fori_loop(0, _ZROWS, zbody, 0)
    base = s * _WR
    for r in range(_WR // _ZROWS):
        pltpu.sync_copy(zbuf, acc.at[pl.ds(base + r * _ZROWS, _ZROWS)])

    @pl.when(s == 15)
    def _ztail():
        pltpu.sync_copy(zbuf.at[pl.ds(0, _N - 16 * _WR)],
                        acc.at[pl.ds(16 * _WR, _N - 16 * _WR)])

    plsc.subcore_barrier()

    for b in range(_RING):
        pltpu.async_copy(h_hbm.at[idx_s.at[b]], rows.at[b], gsems[b])

    def body(gq, carry):
        for b in range(_RING):
            j = gq * _RING + b
            pltpu.make_async_copy(h_hbm.at[idx_s.at[j]], rows.at[b],
                                  gsems[b]).wait()
            pltpu.sync_copy(rows.at[b], acc.at[idx_d.at[j]], add=True)

            @pl.when(j + _RING < _NCH)
            def _fire():
                pltpu.async_copy(h_hbm.at[idx_s.at[j + _RING]],
                                 rows.at[b], gsems[b])
        return carry
    lax.fori_loop(0, _NCH // _RING, body, 0)
    plsc.subcore_barrier()

    pltpu.sync_copy(acc.at[pl.ds(base, _WR)], out_hbm.at[c, pl.ds(base, _WR)])

    @pl.when(s == 15)
    def _wtail():
        pltpu.sync_copy(acc.at[pl.ds(16 * _WR, _N - 16 * _WR)],
                        out_hbm.at[c, pl.ds(16 * _WR, _N - 16 * _WR)])


def _make_sc_scatter():
    mesh = plsc.VectorSubcoreMesh(core_axis_name="c", subcore_axis_name="s")
    return pl.kernel(
        _sc_scatter_body,
        mesh=mesh,
        compiler_params=pltpu.CompilerParams(use_tc_tiling_on_sc=False),
        out_type=jax.ShapeDtypeStruct((2, _N, _D), jnp.float32),
        scratch_types=[
            pltpu.VMEM((_NCH, _K), jnp.int32),
            pltpu.VMEM((_NCH, _K), jnp.int32),
            pltpu.VMEM((_RING, _K, _D), jnp.float32),
            pltpu.VMEM((_ZROWS, _D), jnp.float32),
            pltpu.VMEM_SHARED((_N, _D), jnp.float32),
        ] + [pltpu.SemaphoreType.DMA] * _RING,
    )


_BLK = 400
_NBLK = _N // _BLK


def _mlp1_body(h_ref, p_ref, w1_ref, b1_ref, h1_ref, s1_ref,
               s2_ref, sm2_ref):
    i = pl.program_id(0)
    z = h_ref[...] + p_ref[0] + p_ref[1]
    h1 = jnp.dot(z, w1_ref[...], preferred_element_type=jnp.float32) + b1_ref[...]
    h1_ref[...] = h1

    @pl.when(i == 0)
    def _init():
        s1_ref[...] = jnp.zeros_like(s1_ref)
        s2_ref[...] = jnp.zeros_like(s2_ref)
        sm2_ref[...] = jnp.zeros_like(sm2_ref)

    mb = jnp.mean(h1, axis=0, keepdims=True)
    d = h1 - mb
    s1_ref[...] += mb
    s2_ref[...] += mb * mb
    sm2_ref[...] += jnp.sum(d * d, axis=0, keepdims=True)


def _mlp1(h, part, w1, b1r):
    return pl.pallas_call(
        _mlp1_body,
        grid=(_NBLK,),
        in_specs=[
            pl.BlockSpec((_BLK, _D), lambda i: (i, 0)),
            pl.BlockSpec((2, _BLK, _D), lambda i: (0, i, 0)),
            pl.BlockSpec((_D, _D), lambda i: (0, 0)),
            pl.BlockSpec((1, _D), lambda i: (0, 0)),
        ],
        out_specs=[
            pl.BlockSpec((_BLK, _D), lambda i: (i, 0)),
            pl.BlockSpec((1, _D), lambda i: (0, 0)),
            pl.BlockSpec((1, _D), lambda i: (0, 0)),
            pl.BlockSpec((1, _D), lambda i: (0, 0)),
        ],
        out_shape=[
            jax.ShapeDtypeStruct((_N, _D), jnp.float32),
            jax.ShapeDtypeStruct((1, _D), jnp.float32),
            jax.ShapeDtypeStruct((1, _D), jnp.float32),
            jax.ShapeDtypeStruct((1, _D), jnp.float32),
        ],
    )(h, part, w1, b1r)


def _mlp2_body(h1_ref, s1_ref, s2_ref, sm2_ref, g_ref, be_ref, w2_ref, b2_ref,
               batch_ref, h_ref, pooled_ref):
    i = pl.program_id(0)
    m = s1_ref[...] * (1.0 / _NBLK)
    spread = s2_ref[...] - _NBLK * m * m
    v = (sm2_ref[...] + _BLK * spread) * (1.0 / _N)
    inv = lax.rsqrt(v + 1e-5) * g_ref[...]
    r = jnp.maximum((h1_ref[...] - m) * inv + be_ref[...], 0.0)
    hout = jnp.dot(r, w2_ref[...], preferred_element_type=jnp.float32) + b2_ref[...]
    h_ref[...] = hout

    gid = lax.broadcasted_iota(jnp.int32, (_G, _BLK), 0)
    onehot = (gid == batch_ref[0]).astype(jnp.float32)

    @pl.when(i == 0)
    def _init():
        pooled_ref[...] = jnp.zeros_like(pooled_ref)

    pooled_ref[...] += jnp.dot(onehot, hout, preferred_element_type=jnp.float32, precision=lax.Precision.HIGHEST)


def _mlp2(h1, s1, s2, sm2, gr, ber, w2, b2r, batch3d):
    return pl.pallas_call(
        _mlp2_body,
        grid=(_NBLK,),
        in_specs=[
            pl.BlockSpec((_BLK, _D), lambda i: (i, 0)),
            pl.BlockSpec((1, _D), lambda i: (0, 0)),
            pl.BlockSpec((1, _D), lambda i: (0, 0)),
            pl.BlockSpec((1, _D), lambda i: (0, 0)),
            pl.BlockSpec((1, _D), lambda i: (0, 0)),
            pl.BlockSpec((1, _D), lambda i: (0, 0)),
            pl.BlockSpec((_D, _D), lambda i: (0, 0)),
            pl.BlockSpec((1, _D), lambda i: (0, 0)),
            pl.BlockSpec((1, 1, _BLK), lambda i: (i, 0, 0)),
        ],
        out_specs=[
            pl.BlockSpec((_BLK, _D), lambda i: (i, 0)),
            pl.BlockSpec((_G, _D), lambda i: (0, 0)),
        ],
        out_shape=[
            jax.ShapeDtypeStruct((_N, _D), jnp.float32),
            jax.ShapeDtypeStruct((_G, _D), jnp.float32),
        ],
    )(h1, s1, s2, sm2, gr, ber, w2, b2r, batch3d)


def _head_body(p0_ref, p1_ref, p2_ref, wa_ref, wb_ref, wc_ref, bp1_ref,
               gp_ref, bep_ref, wp2_ref, bp2_ref, out_ref):
    t = (jnp.dot(p0_ref[...], wa_ref[...], preferred_element_type=jnp.float32)
         + jnp.dot(p1_ref[...], wb_ref[...], preferred_element_type=jnp.float32)
         + jnp.dot(p2_ref[...], wc_ref[...], preferred_element_type=jnp.float32)
         + bp1_ref[...])
    m = jnp.mean(t, axis=0, keepdims=True)
    d = t - m
    v = jnp.mean(d * d, axis=0, keepdims=True)
    r = jnp.maximum(d * lax.rsqrt(v + 1e-5) * gp_ref[...] + bep_ref[...], 0.0)
    out_ref[...] = jnp.dot(r, wp2_ref[...], preferred_element_type=jnp.float32) + bp2_ref[...]


def _head(p0, p1, p2, wa, wb, wc, bp1r, gpr, bepr, wp2p, bp2p):
    return pl.pallas_call(
        _head_body,
        out_shape=jax.ShapeDtypeStruct((_G, _D), jnp.float32),
    )(p0, p1, p2, wa, wb, wc, bp1r, gpr, bepr, wp2p, bp2p)


def kernel(x, edge_index, batch, W1_0, b1_0, g_0, be_0, W2_0, b2_0,
           W1_1, b1_1, g_1, be_1, W2_1, b2_1, W1_2, b1_2, g_2, be_2,
           W2_2, b2_2, Wp1, bp1, gp, bep, Wp2, bp2):
    src_r = edge_index[0].reshape(2, 16, _NCH, _K)
    dst_r = edge_index[1].reshape(2, 16, _NCH, _K)
    assert _NCH % _RING == 0
    batch3d = batch.reshape(_NBLK, 1, _BLK)

    sc_scatter = _make_sc_scatter()

    params = [
        (W1_0, b1_0, g_0, be_0, W2_0, b2_0),
        (W1_1, b1_1, g_1, be_1, W2_1, b2_1),
        (W1_2, b1_2, g_2, be_2, W2_2, b2_2),
    ]
    h = x
    pooled = []
    for (w1, b1, g, be, w2, b2) in params:
        part = sc_scatter(h, src_r, dst_r)
        h1, s1, s2, sm2 = _mlp1(h, part, w1, b1.reshape(1, _D))
        h, pool = _mlp2(h1, s1, s2, sm2, g.reshape(1, _D),
                        be.reshape(1, _D), w2, b2.reshape(1, _D),
                        batch3d)
        pooled.append(pool)

    wa = Wp1[0:_D]
    wb = Wp1[_D:2 * _D]
    wc = Wp1[2 * _D:3 * _D]
    wp2p = jnp.pad(Wp2, ((0, 0), (0, _D - _OUT)))
    bp2p = jnp.pad(bp2, (0, _D - _OUT)).reshape(1, _D)
    out = _head(pooled[0], pooled[1], pooled[2], wa, wb, wc,
                bp1.reshape(1, _D), gp.reshape(1, _D), bep.reshape(1, _D),
                wp2p, bp2p)
    return out[:, :_OUT]


# speedup vs baseline: 1.3341x; 1.0381x over previous
import functools

import jax
import jax.numpy as jnp
from jax import lax
from jax.experimental import pallas as pl
from jax.experimental.pallas import tpu as pltpu
from jax.experimental.pallas import tpu_sc as plsc

_N = 10000
_E = 320000
_D = 128
_G = 64
_OUT = 16

_HD = 64
_NCH = 250
_K = 40
_WR = 624
_ZROWS = 24


_RING = 5


def _sc_scatter_body(h_hbm, src_hbm, dst_hbm, out_hbm,
                     idx_s, idx_d, rows, zbuf, acc, *gsems):
    c = lax.axis_index("c")
    s = lax.axis_index("s")

    pltpu.sync_copy(src_hbm.at[c, s], idx_s)
    pltpu.sync_copy(dst_hbm.at[c, s], idx_d)

    def zbody(i, carry):
        for jj in range(_D // 16):
            zbuf[i, pl.ds(jj * 16, 16)] = jnp.zeros((16,), jnp.float32)
        return carry
    lax.fori_loop(0, _ZROWS, zbody, 0)
    base = s * _WR
    for r in range(_WR // _ZROWS):
        pltpu.sync_copy(zbuf, acc.at[pl.ds(base + r * _ZROWS, _ZROWS)])

    @pl.when(s == 15)
    def _ztail():
        pltpu.sync_copy(zbuf.at[pl.ds(0, _N - 16 * _WR)],
                        acc.at[pl.ds(16 * _WR, _N - 16 * _WR)])

    plsc.subcore_barrier()

    for b in range(_RING):
        pltpu.async_copy(h_hbm.at[idx_s.at[b]], rows.at[b], gsems[b])

    def body(gq, carry):
        for b in range(_RING):
            j = gq * _RING + b
            pltpu.make_async_copy(h_hbm.at[idx_s.at[j]], rows.at[b],
                                  gsems[b]).wait()
            pltpu.sync_copy(rows.at[b], acc.at[idx_d.at[j]], add=True)

            @pl.when(j + _RING < _NCH)
            def _fire():
                pltpu.async_copy(h_hbm.at[idx_s.at[j + _RING]],
                                 rows.at[b], gsems[b])
        return carry
    lax.fori_loop(0, _NCH // _RING, body, 0)
    plsc.subcore_barrier()

    pltpu.sync_copy(acc.at[pl.ds(base, _WR)], out_hbm.at[c, pl.ds(base, _WR)])

    @pl.when(s == 15)
    def _wtail():
        pltpu.sync_copy(acc.at[pl.ds(16 * _WR, _N - 16 * _WR)],
                        out_hbm.at[c, pl.ds(16 * _WR, _N - 16 * _WR)])


def _make_sc_scatter():
    mesh = plsc.VectorSubcoreMesh(core_axis_name="c", subcore_axis_name="s")
    return pl.kernel(
        _sc_scatter_body,
        mesh=mesh,
        compiler_params=pltpu.CompilerParams(use_tc_tiling_on_sc=False),
        out_type=jax.ShapeDtypeStruct((2, _N, _D), jnp.float32),
        scratch_types=[
            pltpu.VMEM((_NCH, _K), jnp.int32),
            pltpu.VMEM((_NCH, _K), jnp.int32),
            pltpu.VMEM((_RING, _K, _D), jnp.float32),
            pltpu.VMEM((_ZROWS, _D), jnp.float32),
            pltpu.VMEM_SHARED((_N, _D), jnp.float32),
        ] + [pltpu.SemaphoreType.DMA] * _RING,
    )


_BLK = 400
_NBLK = _N // _BLK


def _mlp_fused_body(h_ref, p_ref, w1_ref, b1_ref, g_ref, be_ref, w2_ref,
                    b2_ref, batch_ref, hout_ref, pooled_ref,
                    h1s, s1s, s2s, sm2s):
    p = pl.program_id(0)
    j = pl.program_id(1)

    @pl.when(p == 0)
    def _pass0():
        z = h_ref[...] + p_ref[0] + p_ref[1]
        h1 = jnp.dot(z, w1_ref[...], preferred_element_type=jnp.float32) + b1_ref[...]
        h1s[pl.ds(j * _BLK, _BLK), :] = h1

        @pl.when(j == 0)
        def _init():
            s1s[...] = jnp.zeros_like(s1s)
            s2s[...] = jnp.zeros_like(s2s)
            sm2s[...] = jnp.zeros_like(sm2s)

        mb = jnp.mean(h1, axis=0, keepdims=True)
        d = h1 - mb
        s1s[...] += mb
        s2s[...] += mb * mb
        sm2s[...] += jnp.sum(d * d, axis=0, keepdims=True)

    @pl.when(p == 1)
    def _pass1():
        m = s1s[...] * (1.0 / _NBLK)
        spread = s2s[...] - _NBLK * m * m
        v = (sm2s[...] + _BLK * spread) * (1.0 / _N)
        inv = lax.rsqrt(v + 1e-5) * g_ref[...]
        h1 = h1s[pl.ds(j * _BLK, _BLK), :]
        r = jnp.maximum((h1 - m) * inv + be_ref[...], 0.0)
        hout = jnp.dot(r, w2_ref[...], preferred_element_type=jnp.float32) + b2_ref[...]
        hout_ref[...] = hout

        gid = lax.broadcasted_iota(jnp.int32, (_G, _BLK), 0)
        onehot = (gid == batch_ref[0]).astype(jnp.float32)

        @pl.when(j == 0)
        def _initp():
            pooled_ref[...] = jnp.zeros_like(pooled_ref)

        pooled_ref[...] += jnp.dot(onehot, hout,
                                   preferred_element_type=jnp.float32,
                                   precision=lax.Precision.HIGHEST)


def _mlp_fused(h, part, w1, b1r, gr, ber, w2, b2r, batch3d):
    return pl.pallas_call(
        _mlp_fused_body,
        grid=(2, _NBLK),
        in_specs=[
            pl.BlockSpec((_BLK, _D), lambda p, j: ((1 - p) * j, 0)),
            pl.BlockSpec((2, _BLK, _D), lambda p, j: (0, (1 - p) * j, 0)),
            pl.BlockSpec((_D, _D), lambda p, j: (0, 0)),
            pl.BlockSpec((1, _D), lambda p, j: (0, 0)),
            pl.BlockSpec((1, _D), lambda p, j: (0, 0)),
            pl.BlockSpec((1, _D), lambda p, j: (0, 0)),
            pl.BlockSpec((_D, _D), lambda p, j: (0, 0)),
            pl.BlockSpec((1, _D), lambda p, j: (0, 0)),
            pl.BlockSpec((1, 1, _BLK), lambda p, j: (p * j, 0, 0)),
        ],
        out_specs=[
            pl.BlockSpec((_BLK, _D), lambda p, j: (p * j, 0)),
            pl.BlockSpec((_G, _D), lambda p, j: (0, 0)),
        ],
        out_shape=[
            jax.ShapeDtypeStruct((_N, _D), jnp.float32),
            jax.ShapeDtypeStruct((_G, _D), jnp.float32),
        ],
        scratch_shapes=[
            pltpu.VMEM((_N, _D), jnp.float32),
            pltpu.VMEM((1, _D), jnp.float32),
            pltpu.VMEM((1, _D), jnp.float32),
            pltpu.VMEM((1, _D), jnp.float32),
        ],
    )(h, part, w1, b1r, gr, ber, w2, b2r, batch3d)


def _head_body(p0_ref, p1_ref, p2_ref, wa_ref, wb_ref, wc_ref, bp1_ref,
               gp_ref, bep_ref, wp2_ref, bp2_ref, out_ref):
    t = (jnp.dot(p0_ref[...], wa_ref[...], preferred_element_type=jnp.float32)
         + jnp.dot(p1_ref[...], wb_ref[...], preferred_element_type=jnp.float32)
         + jnp.dot(p2_ref[...], wc_ref[...], preferred_element_type=jnp.float32)
         + bp1_ref[...])
    m = jnp.mean(t, axis=0, keepdims=True)
    d = t - m
    v = jnp.mean(d * d, axis=0, keepdims=True)
    r = jnp.maximum(d * lax.rsqrt(v + 1e-5) * gp_ref[...] + bep_ref[...], 0.0)
    out_ref[...] = jnp.dot(r, wp2_ref[...], preferred_element_type=jnp.float32) + bp2_ref[...]


def _head(p0, p1, p2, wa, wb, wc, bp1r, gpr, bepr, wp2p, bp2p):
    return pl.pallas_call(
        _head_body,
        out_shape=jax.ShapeDtypeStruct((_G, _D), jnp.float32),
    )(p0, p1, p2, wa, wb, wc, bp1r, gpr, bepr, wp2p, bp2p)


def kernel(x, edge_index, batch, W1_0, b1_0, g_0, be_0, W2_0, b2_0,
           W1_1, b1_1, g_1, be_1, W2_1, b2_1, W1_2, b1_2, g_2, be_2,
           W2_2, b2_2, Wp1, bp1, gp, bep, Wp2, bp2):
    src_r = edge_index[0].reshape(2, 16, _NCH, _K)
    dst_r = edge_index[1].reshape(2, 16, _NCH, _K)
    assert _NCH % _RING == 0
    batch3d = batch.reshape(_NBLK, 1, _BLK)

    sc_scatter = _make_sc_scatter()

    params = [
        (W1_0, b1_0, g_0, be_0, W2_0, b2_0),
        (W1_1, b1_1, g_1, be_1, W2_1, b2_1),
        (W1_2, b1_2, g_2, be_2, W2_2, b2_2),
    ]
    h = x
    pooled = []
    for (w1, b1, g, be, w2, b2) in params:
        part = sc_scatter(h, src_r, dst_r)
        h, pool = _mlp_fused(h, part, w1, b1.reshape(1, _D),
                             g.reshape(1, _D), be.reshape(1, _D),
                             w2, b2.reshape(1, _D), batch3d)
        pooled.append(pool)

    wa = Wp1[0:_D]
    wb = Wp1[_D:2 * _D]
    wc = Wp1[2 * _D:3 * _D]
    wp2p = jnp.pad(Wp2, ((0, 0), (0, _D - _OUT)))
    bp2p = jnp.pad(bp2, (0, _D - _OUT)).reshape(1, _D)
    out = _head(pooled[0], pooled[1], pooled[2], wa, wb, wc,
                bp1.reshape(1, _D), gp.reshape(1, _D), bep.reshape(1, _D),
                wp2p, bp2p)
    return out[:, :_OUT]
